# baseline (speedup 1.0000x reference)
import functools

import jax
import jax.numpy as jnp
from jax import lax
from jax.experimental import pallas as pl
from jax.experimental.pallas import tpu as pltpu
from jax.experimental.pallas import tpu_sc as plsc

LANES = 16


def _lane_bcast(vec, lane):
    idx = jnp.full((LANES,), lane, jnp.int32)
    return vec.at[idx].get(mode="promise_in_bounds")


def _lane_reduce(vec, op):
    idx = lax.iota(jnp.int32, LANES)
    for sh in (8, 4, 2, 1):
        shuf = vec.at[idx ^ sh].get(mode="promise_in_bounds")
        vec = op(vec, shuf)
    return vec


def _make_kernel(B, L, D, V, num_workers, cb):
    rows_per_worker = B // num_workers
    n_chunks = rows_per_worker // cb
    assert n_chunks % 2 == 0
    n_idx = cb * L
    d_regs = D // LANES
    offs = (0, 16, 32, 34)

    mesh = plsc.VectorSubcoreMesh(core_axis_name="c", subcore_axis_name="s")

    @functools.partial(
        pl.kernel,
        out_type=jax.ShapeDtypeStruct((B, D), jnp.float32),
        mesh=mesh,
        scratch_types=[
            pltpu.VMEM((cb, L), jnp.int32),
            pltpu.VMEM((cb, L), jnp.int32),
            pltpu.VMEM((cb, L), jnp.float32),
            pltpu.VMEM((cb, L), jnp.float32),
            pltpu.VMEM((n_idx, D), jnp.bfloat16),
            pltpu.VMEM((n_idx, D), jnp.bfloat16),
            pltpu.VMEM((cb, D), jnp.float32),
            pltpu.VMEM((cb, D), jnp.float32),
            pltpu.SemaphoreType.DMA,
            pltpu.SemaphoreType.DMA,
        ],
        compiler_params=pltpu.CompilerParams(use_tc_tiling_on_sc=False, needs_layout_passes=False),
    )
    def kern(ids, weights, table, out, id0_v, id1_v, w0_v, w1_v,
             emb0_v, emb1_v, p_v, out_v, sem0, sem1):
        nc = mesh.num_cores
        wid = lax.axis_index("s") * nc + lax.axis_index("c")
        row0 = wid * rows_per_worker
        sems = (sem0, sem1)
        id_b = (id0_v, id1_v)
        w_b = (w0_v, w1_v)
        emb_b = (emb0_v, emb1_v)

        def fetch(c, b):
            rbase = row0 + c * cb
            pltpu.sync_copy(ids.at[pl.ds(rbase, cb)], id_b[b])
            pltpu.sync_copy(weights.at[pl.ds(rbase, cb)], w_b[b])
            for r in range(cb):
                pltpu.async_copy(table.at[id_b[b].at[r]],
                                 emb_b[b].at[pl.ds(r * L, L)], sems[b])

        def wait_gathers(b):
            for r in range(cb):
                pltpu.make_async_copy(table.at[id_b[b].at[r]],
                                      emb_b[b].at[pl.ds(r * L, L)],
                                      sems[b]).wait()

        def softmax_row(r, b):
            iv = [id_b[b][r, pl.ds(o, LANES)] for o in offs]
            wv = [w_b[b][r, pl.ds(o, LANES)] for o in offs]
            mw = [jnp.where(iv[k] == 0, jnp.float32(-1e9), wv[k])
                  for k in range(4)]
            lane = lax.iota(jnp.int32, LANES)
            mw[2] = jnp.where(lane < 2, mw[2], jnp.float32(-jnp.inf))
            mx = jnp.maximum(jnp.maximum(mw[0], mw[1]),
                             jnp.maximum(mw[2], mw[3]))
            m = _lane_reduce(mx, jnp.maximum)
            ev = [jnp.exp(mw[k] - m) for k in range(4)]
            z = _lane_reduce(ev[0] + ev[1] + ev[2] + ev[3], jnp.add)
            inv_z = jnp.float32(1.0) / z
            p0 = ev[0] * inv_z
            p1 = ev[1] * inv_z
            sel = lane < 2
            shuf3 = ev[3].at[(lane - 2) & 15].get(mode="promise_in_bounds")
            p2 = jnp.where(sel, ev[2], shuf3) * inv_z
            p3 = jnp.where(sel, shuf3, jnp.float32(0.0)) * inv_z
            p_v[r, pl.ds(0, LANES)] = p0
            p_v[r, pl.ds(LANES, LANES)] = p1
            p_v[r, pl.ds(2 * LANES, LANES)] = p2
            p_v[r, pl.ds(3 * LANES, LANES)] = p3

        def compute(c, b):
            rbase = row0 + c * cb
            for r in range(cb):
                softmax_row(r, b)

            def row_body(r, _):
                pv = [p_v[r, pl.ds(k * LANES, LANES)] for k in range(4)]
                n_ch = D // (2 * LANES)
                acc_e = [jnp.zeros((LANES,), jnp.float32)
                         for _ in range(n_ch)]
                acc_o = [jnp.zeros((LANES,), jnp.float32)
                         for _ in range(n_ch)]
                for l in range(L):
                    s = _lane_bcast(pv[l // LANES], l % LANES)
                    for c in range(n_ch):
                        ab = emb_b[b][r * L + l,
                                      pl.ds(c * 2 * LANES, 2 * LANES)]
                        ea, eb = plsc.unpack(
                            ab, format=plsc.PackFormat.INTERLEAVED)
                        acc_e[c] = acc_e[c] + s * ea
                        acc_o[c] = acc_o[c] + s * eb
                lane = lax.iota(jnp.int32, LANES)
                even = (lane & 1) == 0
                for k in range(d_regs):
                    c, h = k // 2, k % 2
                    gidx = h * 8 + (lane >> 1)
                    ge = acc_e[c].at[gidx].get(mode="promise_in_bounds")
                    go = acc_o[c].at[gidx].get(mode="promise_in_bounds")
                    out_v[r, pl.ds(k * LANES, LANES)] = jnp.where(
                        even, ge, go)
                return 0

            lax.fori_loop(0, cb, row_body, 0, unroll=False)
            pltpu.sync_copy(out_v, out.at[pl.ds(rbase, cb)])

        fetch(0, 0)
        fetch(1, 1)

        def pair_body(gp, _):
            for bb in range(2):
                c = gp * 2 + bb
                wait_gathers(bb)
                compute(c, bb)

                @pl.when(c + 2 < n_chunks)
                def _():
                    fetch(c + 2, bb)
            return 0

        lax.fori_loop(0, n_chunks // 2, pair_body, 0, unroll=False)

    return kern


def kernel(ids, weights, table):
    B, L = ids.shape
    V, D = table.shape
    ids = ids.astype(jnp.int32)
    info = plsc.get_sparse_core_info()
    num_workers = info.num_cores * info.num_subcores
    kern = _make_kernel(B, L, D, V, num_workers, cb=16)
    return kern(ids, weights, table.astype(jnp.bfloat16))


# speedup vs baseline: 1.2960x; 1.2960x over previous
import functools

import jax
import jax.numpy as jnp
from jax import lax
from jax.experimental import pallas as pl
from jax.experimental.pallas import tpu as pltpu
from jax.experimental.pallas import tpu_sc as plsc

LANES = 16


def _lane_bcast(vec, lane):
    idx = jnp.full((LANES,), lane, jnp.int32)
    return vec.at[idx].get(mode="promise_in_bounds")


def _lane_reduce(vec, op):
    idx = lax.iota(jnp.int32, LANES)
    for sh in (8, 4, 2, 1):
        shuf = vec.at[idx ^ sh].get(mode="promise_in_bounds")
        vec = op(vec, shuf)
    return vec


def _make_kernel(B, L, D, V, num_workers, cb):
    rows_per_worker = B // num_workers
    n_chunks = rows_per_worker // cb
    assert n_chunks % 2 == 0
    n_idx = cb * L
    d_regs = D // LANES
    offs = (0, 16, 32, 34)

    mesh = plsc.VectorSubcoreMesh(core_axis_name="c", subcore_axis_name="s")

    @functools.partial(
        pl.kernel,
        out_type=jax.ShapeDtypeStruct((B, D), jnp.float32),
        mesh=mesh,
        scratch_types=[
            pltpu.VMEM((cb, L), jnp.int32),
            pltpu.VMEM((cb, L), jnp.int32),
            pltpu.VMEM((cb, L), jnp.float32),
            pltpu.VMEM((cb, L), jnp.float32),
            pltpu.VMEM((n_idx, D), jnp.float32),
            pltpu.VMEM((n_idx, D), jnp.float32),
            pltpu.VMEM((cb, D), jnp.float32),
            pltpu.VMEM((cb, D), jnp.float32),
            pltpu.SemaphoreType.DMA,
            pltpu.SemaphoreType.DMA,
        ],
        compiler_params=pltpu.CompilerParams(use_tc_tiling_on_sc=False),
    )
    def kern(ids, weights, table, out, id0_v, id1_v, w0_v, w1_v,
             emb0_v, emb1_v, p_v, out_v, sem0, sem1):
        nc = mesh.num_cores
        wid = lax.axis_index("s") * nc + lax.axis_index("c")
        row0 = wid * rows_per_worker
        sems = (sem0, sem1)
        id_b = (id0_v, id1_v)
        w_b = (w0_v, w1_v)
        emb_b = (emb0_v, emb1_v)

        def fetch(c, b):
            rbase = row0 + c * cb
            pltpu.sync_copy(ids.at[pl.ds(rbase, cb)], id_b[b])
            pltpu.sync_copy(weights.at[pl.ds(rbase, cb)], w_b[b])
            for r in range(cb):
                pltpu.async_copy(table.at[id_b[b].at[r]],
                                 emb_b[b].at[pl.ds(r * L, L)], sems[b])

        def wait_gathers(b):
            for r in range(cb):
                pltpu.make_async_copy(table.at[id_b[b].at[r]],
                                      emb_b[b].at[pl.ds(r * L, L)],
                                      sems[b]).wait()

        def softmax_row(r, b):
            iv = [id_b[b][r, pl.ds(o, LANES)] for o in offs]
            wv = [w_b[b][r, pl.ds(o, LANES)] for o in offs]
            mw = [jnp.where(iv[k] == 0, jnp.float32(-1e9), wv[k])
                  for k in range(4)]
            lane = lax.iota(jnp.int32, LANES)
            mw[2] = jnp.where(lane < 2, mw[2], jnp.float32(-jnp.inf))
            mx = jnp.maximum(jnp.maximum(mw[0], mw[1]),
                             jnp.maximum(mw[2], mw[3]))
            m = _lane_reduce(mx, jnp.maximum)
            ev = [jnp.exp(mw[k] - m) for k in range(4)]
            z = _lane_reduce(ev[0] + ev[1] + ev[2] + ev[3], jnp.add)
            inv_z = jnp.float32(1.0) / z
            p0 = ev[0] * inv_z
            p1 = ev[1] * inv_z
            sel = lane < 2
            shuf3 = ev[3].at[(lane - 2) & 15].get(mode="promise_in_bounds")
            p2 = jnp.where(sel, ev[2], shuf3) * inv_z
            p3 = jnp.where(sel, shuf3, jnp.float32(0.0)) * inv_z
            p_v[r, pl.ds(0, LANES)] = p0
            p_v[r, pl.ds(LANES, LANES)] = p1
            p_v[r, pl.ds(2 * LANES, LANES)] = p2
            p_v[r, pl.ds(3 * LANES, LANES)] = p3

        def compute(c, b):
            rbase = row0 + c * cb
            for r in range(cb):
                softmax_row(r, b)

            def row_body(r, _):
                pv = [p_v[r, pl.ds(k * LANES, LANES)] for k in range(4)]
                acc = [jnp.zeros((LANES,), jnp.float32)
                       for _ in range(d_regs)]
                for l in range(L):
                    s = _lane_bcast(pv[l // LANES], l % LANES)
                    for k in range(d_regs):
                        acc[k] = acc[k] + s * emb_b[b][
                            r * L + l, pl.ds(k * LANES, LANES)]
                for k in range(d_regs):
                    out_v[r, pl.ds(k * LANES, LANES)] = acc[k]
                return 0

            lax.fori_loop(0, cb, row_body, 0, unroll=False)
            pltpu.sync_copy(out_v, out.at[pl.ds(rbase, cb)])

        fetch(0, 0)
        fetch(1, 1)

        def pair_body(gp, _):
            for bb in range(2):
                c = gp * 2 + bb
                wait_gathers(bb)
                compute(c, bb)

                @pl.when(c + 2 < n_chunks)
                def _():
                    fetch(c + 2, bb)
            return 0

        lax.fori_loop(0, n_chunks // 2, pair_body, 0, unroll=False)

    return kern


def kernel(ids, weights, table):
    B, L = ids.shape
    V, D = table.shape
    ids = ids.astype(jnp.int32)
    info = plsc.get_sparse_core_info()
    num_workers = info.num_cores * info.num_subcores
    kern = _make_kernel(B, L, D, V, num_workers, cb=16)
    return kern(ids, weights, table)


# speedup vs baseline: 1.3030x; 1.0054x over previous
import functools

import jax
import jax.numpy as jnp
from jax import lax
from jax.experimental import pallas as pl
from jax.experimental.pallas import tpu as pltpu
from jax.experimental.pallas import tpu_sc as plsc

LANES = 16


def _lane_bcast(vec, lane):
    idx = jnp.full((LANES,), lane, jnp.int32)
    return vec.at[idx].get(mode="promise_in_bounds")


def _lane_reduce(vec, op):
    idx = lax.iota(jnp.int32, LANES)
    for sh in (8, 4, 2, 1):
        shuf = vec.at[idx ^ sh].get(mode="promise_in_bounds")
        vec = op(vec, shuf)
    return vec


def _make_kernel(B, L, D, V, num_workers, cb):
    rows_per_worker = B // num_workers
    n_chunks = rows_per_worker // cb
    assert n_chunks % 2 == 0
    n_idx = cb * L
    d_regs = D // LANES
    offs = (0, 16, 32, 34)

    mesh = plsc.VectorSubcoreMesh(core_axis_name="c", subcore_axis_name="s")

    @functools.partial(
        pl.kernel,
        out_type=jax.ShapeDtypeStruct((B, D), jnp.float32),
        mesh=mesh,
        scratch_types=[
            pltpu.VMEM((cb, L), jnp.int32),
            pltpu.VMEM((cb, L), jnp.int32),
            pltpu.VMEM((cb, L), jnp.float32),
            pltpu.VMEM((cb, L), jnp.float32),
            pltpu.VMEM((n_idx, D), jnp.float32),
            pltpu.VMEM((n_idx, D), jnp.float32),
            pltpu.VMEM((cb, D), jnp.float32),
            pltpu.VMEM((cb, D), jnp.float32),
            pltpu.VMEM((cb, D), jnp.float32),
            pltpu.SemaphoreType.DMA,
            pltpu.SemaphoreType.DMA,
            pltpu.SemaphoreType.DMA,
            pltpu.SemaphoreType.DMA,
        ],
        compiler_params=pltpu.CompilerParams(use_tc_tiling_on_sc=False),
    )
    def kern(ids, weights, table, out, id0_v, id1_v, w0_v, w1_v,
             emb0_v, emb1_v, p_v, out0_v, out1_v, sem0, sem1,
             semo0, semo1):
        nc = mesh.num_cores
        wid = lax.axis_index("s") * nc + lax.axis_index("c")
        row0 = wid * rows_per_worker
        sems = (sem0, sem1)
        id_b = (id0_v, id1_v)
        w_b = (w0_v, w1_v)
        emb_b = (emb0_v, emb1_v)
        out_b = (out0_v, out1_v)
        semo = (semo0, semo1)

        def fetch(c, b):
            rbase = row0 + c * cb
            pltpu.sync_copy(ids.at[pl.ds(rbase, cb)], id_b[b])
            pltpu.sync_copy(weights.at[pl.ds(rbase, cb)], w_b[b])
            for r in range(cb):
                pltpu.async_copy(table.at[id_b[b].at[r]],
                                 emb_b[b].at[pl.ds(r * L, L)], sems[b])

        def wait_gathers(b):
            for r in range(cb):
                pltpu.make_async_copy(table.at[id_b[b].at[r]],
                                      emb_b[b].at[pl.ds(r * L, L)],
                                      sems[b]).wait()

        def softmax_row(r, b):
            iv = [id_b[b][r, pl.ds(o, LANES)] for o in offs]
            wv = [w_b[b][r, pl.ds(o, LANES)] for o in offs]
            mw = [jnp.where(iv[k] == 0, jnp.float32(-1e9), wv[k])
                  for k in range(4)]
            lane = lax.iota(jnp.int32, LANES)
            mw[2] = jnp.where(lane < 2, mw[2], jnp.float32(-jnp.inf))
            mx = jnp.maximum(jnp.maximum(mw[0], mw[1]),
                             jnp.maximum(mw[2], mw[3]))
            m = _lane_reduce(mx, jnp.maximum)
            ev = [jnp.exp(mw[k] - m) for k in range(4)]
            z = _lane_reduce(ev[0] + ev[1] + ev[2] + ev[3], jnp.add)
            inv_z = jnp.float32(1.0) / z
            p0 = ev[0] * inv_z
            p1 = ev[1] * inv_z
            sel = lane < 2
            shuf3 = ev[3].at[(lane - 2) & 15].get(mode="promise_in_bounds")
            p2 = jnp.where(sel, ev[2], shuf3) * inv_z
            p3 = jnp.where(sel, shuf3, jnp.float32(0.0)) * inv_z
            p_v[r, pl.ds(0, LANES)] = p0
            p_v[r, pl.ds(LANES, LANES)] = p1
            p_v[r, pl.ds(2 * LANES, LANES)] = p2
            p_v[r, pl.ds(3 * LANES, LANES)] = p3

        def compute(c, b):
            rbase = row0 + c * cb
            for r in range(cb):
                softmax_row(r, b)

            @pl.when(c >= 2)
            def _():
                pltpu.make_async_copy(
                    out_b[b], out.at[pl.ds(row0, cb)], semo[b]).wait()

            out_v = out_b[b]

            def row_body(r, _):
                pv = [p_v[r, pl.ds(k * LANES, LANES)] for k in range(4)]
                acc = [jnp.zeros((LANES,), jnp.float32)
                       for _ in range(d_regs)]
                for l in range(L):
                    s = _lane_bcast(pv[l // LANES], l % LANES)
                    for k in range(d_regs):
                        acc[k] = acc[k] + s * emb_b[b][
                            r * L + l, pl.ds(k * LANES, LANES)]
                for k in range(d_regs):
                    out_v[r, pl.ds(k * LANES, LANES)] = acc[k]
                return 0

            lax.fori_loop(0, cb, row_body, 0, unroll=False)
            pltpu.async_copy(out_v, out.at[pl.ds(rbase, cb)], semo[b])

        fetch(0, 0)
        fetch(1, 1)

        def pair_body(gp, _):
            for bb in range(2):
                c = gp * 2 + bb
                wait_gathers(bb)
                compute(c, bb)

                @pl.when(c + 2 < n_chunks)
                def _():
                    fetch(c + 2, bb)
            return 0

        lax.fori_loop(0, n_chunks // 2, pair_body, 0, unroll=False)
        for bb in range(2):
            pltpu.make_async_copy(
                out_b[bb], out.at[pl.ds(row0, cb)], semo[bb]).wait()

    return kern


def kernel(ids, weights, table):
    B, L = ids.shape
    V, D = table.shape
    ids = ids.astype(jnp.int32)
    info = plsc.get_sparse_core_info()
    num_workers = info.num_cores * info.num_subcores
    kern = _make_kernel(B, L, D, V, num_workers, cb=16)
    return kern(ids, weights, table)


# speedup vs baseline: 1.3036x; 1.0004x over previous
import functools

import jax
import jax.numpy as jnp
from jax import lax
from jax.experimental import pallas as pl
from jax.experimental.pallas import tpu as pltpu
from jax.experimental.pallas import tpu_sc as plsc

LANES = 16


def _lane_bcast(vec, lane):
    idx = jnp.full((LANES,), lane, jnp.int32)
    return vec.at[idx].get(mode="promise_in_bounds")


def _lane_reduce(vec, op):
    idx = lax.iota(jnp.int32, LANES)
    for sh in (8, 4, 2, 1):
        shuf = vec.at[idx ^ sh].get(mode="promise_in_bounds")
        vec = op(vec, shuf)
    return vec


def _make_kernel(B, L, D, V, num_workers, cb):
    rows_per_worker = B // num_workers
    n_chunks = rows_per_worker // cb
    assert n_chunks % 2 == 0
    n_idx = cb * L
    d_regs = D // LANES
    offs = (0, 16, 32, 34)

    mesh = plsc.VectorSubcoreMesh(core_axis_name="c", subcore_axis_name="s")

    @functools.partial(
        pl.kernel,
        out_type=jax.ShapeDtypeStruct((B, D), jnp.float32),
        mesh=mesh,
        scratch_types=[
            pltpu.VMEM((cb, L), jnp.int32),
            pltpu.VMEM((cb, L), jnp.int32),
            pltpu.VMEM((cb, L), jnp.float32),
            pltpu.VMEM((cb, L), jnp.float32),
            pltpu.VMEM((n_idx, D), jnp.float32),
            pltpu.VMEM((n_idx, D), jnp.float32),
            pltpu.VMEM((cb, D), jnp.float32),
            pltpu.VMEM((cb, D), jnp.float32),
            pltpu.VMEM((cb, D), jnp.float32),
            pltpu.SemaphoreType.DMA,
            pltpu.SemaphoreType.DMA,
            pltpu.SemaphoreType.DMA,
            pltpu.SemaphoreType.DMA,
        ],
        compiler_params=pltpu.CompilerParams(use_tc_tiling_on_sc=False),
    )
    def kern(ids, weights, table, out, id0_v, id1_v, w0_v, w1_v,
             emb0_v, emb1_v, p_v, out0_v, out1_v, sem0, sem1,
             semo0, semo1):
        nc = mesh.num_cores
        wid = lax.axis_index("s") * nc + lax.axis_index("c")
        row0 = wid * rows_per_worker
        sems = (sem0, sem1)
        id_b = (id0_v, id1_v)
        w_b = (w0_v, w1_v)
        emb_b = (emb0_v, emb1_v)
        out_b = (out0_v, out1_v)
        semo = (semo0, semo1)

        def fetch(c, b):
            rbase = row0 + c * cb
            pltpu.sync_copy(ids.at[pl.ds(rbase, cb)], id_b[b])
            pltpu.sync_copy(weights.at[pl.ds(rbase, cb)], w_b[b])
            for r in range(cb):
                pltpu.async_copy(table.at[id_b[b].at[r]],
                                 emb_b[b].at[pl.ds(r * L, L)], sems[b])

        def wait_gathers(b):
            for r in range(cb):
                pltpu.make_async_copy(table.at[id_b[b].at[r]],
                                      emb_b[b].at[pl.ds(r * L, L)],
                                      sems[b]).wait()

        def softmax_row(r, b):
            iv = [id_b[b][r, pl.ds(o, LANES)] for o in offs]
            wv = [w_b[b][r, pl.ds(o, LANES)] for o in offs]
            mw = [jnp.where(iv[k] == 0, jnp.float32(-1e9), wv[k])
                  for k in range(4)]
            lane = lax.iota(jnp.int32, LANES)
            mw[2] = jnp.where(lane < 2, mw[2], jnp.float32(-jnp.inf))
            mx = jnp.maximum(jnp.maximum(mw[0], mw[1]),
                             jnp.maximum(mw[2], mw[3]))
            m = _lane_reduce(mx, jnp.maximum)
            ev = [jnp.exp(mw[k] - m) for k in range(4)]
            z = _lane_reduce(ev[0] + ev[1] + ev[2] + ev[3], jnp.add)
            inv_z = jnp.float32(1.0) / z
            p0 = ev[0] * inv_z
            p1 = ev[1] * inv_z
            sel = lane < 2
            shuf3 = ev[3].at[(lane - 2) & 15].get(mode="promise_in_bounds")
            p2 = jnp.where(sel, ev[2], shuf3) * inv_z
            p3 = jnp.where(sel, shuf3, jnp.float32(0.0)) * inv_z
            p_v[r, pl.ds(0, LANES)] = p0
            p_v[r, pl.ds(LANES, LANES)] = p1
            p_v[r, pl.ds(2 * LANES, LANES)] = p2
            p_v[r, pl.ds(3 * LANES, LANES)] = p3

        def compute(c, b):
            rbase = row0 + c * cb
            @pl.when(c >= 2)
            def _():
                pltpu.make_async_copy(
                    out_b[b], out.at[pl.ds(row0, cb)], semo[b]).wait()

            out_v = out_b[b]

            def row_body(r, _):
                pv = [p_v[r, pl.ds(k * LANES, LANES)] for k in range(4)]
                acc = [jnp.zeros((LANES,), jnp.float32)
                       for _ in range(d_regs)]
                for l in range(L):
                    s = _lane_bcast(pv[l // LANES], l % LANES)
                    for k in range(d_regs):
                        acc[k] = acc[k] + s * emb_b[b][
                            r * L + l, pl.ds(k * LANES, LANES)]
                for k in range(d_regs):
                    out_v[r, pl.ds(k * LANES, LANES)] = acc[k]
                return 0

            lax.fori_loop(0, cb, row_body, 0, unroll=False)
            pltpu.async_copy(out_v, out.at[pl.ds(rbase, cb)], semo[b])

        fetch(0, 0)
        fetch(1, 1)

        def pair_body(gp, _):
            for bb in range(2):
                c = gp * 2 + bb
                for r in range(cb):
                    softmax_row(r, bb)
                wait_gathers(bb)
                compute(c, bb)

                @pl.when(c + 2 < n_chunks)
                def _():
                    fetch(c + 2, bb)
            return 0

        lax.fori_loop(0, n_chunks // 2, pair_body, 0, unroll=False)
        for bb in range(2):
            pltpu.make_async_copy(
                out_b[bb], out.at[pl.ds(row0, cb)], semo[bb]).wait()

    return kern


def kernel(ids, weights, table):
    B, L = ids.shape
    V, D = table.shape
    ids = ids.astype(jnp.int32)
    info = plsc.get_sparse_core_info()
    num_workers = info.num_cores * info.num_subcores
    kern = _make_kernel(B, L, D, V, num_workers, cb=16)
    return kern(ids, weights, table)
